# baseline (device time: 40660 ns/iter reference)
import jax
import jax.numpy as jnp
from jax import lax
from jax.experimental import pallas as pl
from jax.experimental.pallas import tpu as pltpu

N_DEV = 8
E_PER = 2
N_EXP = N_DEV * E_PER
T = 256
D = 128
H = 256


def kernel(x, router_W, route_idx, expert_W):
    def body(x_ref, rw_ref, ridx_ref, ew_ref, out_ref,
             comm_ref, send_sems, recv_sems):
        my_pos = lax.axis_index("i")
        left = lax.rem(my_pos - 1 + N_DEV, N_DEV)
        right = lax.rem(my_pos + 1, N_DEV)

        barrier_sem = pltpu.get_barrier_semaphore()
        for nbr in (left, right):
            pl.semaphore_signal(
                barrier_sem, inc=1,
                device_id=(nbr,), device_id_type=pl.DeviceIdType.MESH,
            )
        pl.semaphore_wait(barrier_sem, 2)

        xv = x_ref[:, :]
        scores = jnp.dot(xv, rw_ref[:, :], preferred_element_type=jnp.float32)
        m = jnp.max(scores, axis=1, keepdims=True)
        p = jnp.exp(scores - m)
        p = p / jnp.sum(p, axis=1, keepdims=True)
        e0 = ridx_ref[:, 0:1]
        e1 = ridx_ref[:, 1:2]
        iota16 = lax.broadcasted_iota(jnp.int32, (T, N_EXP), 1)
        g0 = jnp.sum(jnp.where(iota16 == e0, p, 0.0), axis=1, keepdims=True)
        g1 = jnp.sum(jnp.where(iota16 == e1, p, 0.0), axis=1, keepdims=True)
        gs = g0 + g1
        g0n = g0 / gs
        g1n = g1 / gs

        def chunk_compute(w_pair, origin, acc):
            oa = E_PER * origin
            ca = g0n * (e0 == oa).astype(jnp.float32) \
                + g1n * (e1 == oa).astype(jnp.float32)
            cb = g0n * (e0 == oa + 1).astype(jnp.float32) \
                + g1n * (e1 == oa + 1).astype(jnp.float32)
            xcat = jnp.concatenate([ca * xv, cb * xv], axis=1)
            wcat = w_pair.reshape(E_PER * D, H)
            return acc + jnp.dot(xcat, wcat, preferred_element_type=jnp.float32)

        acc = chunk_compute(ew_ref[:, :, :], my_pos,
                            jnp.zeros((T, H), jnp.float32))

        for h in range(N_DEV - 1):
            src = ew_ref if h == 0 else comm_ref.at[h - 1]
            rdma = pltpu.make_async_remote_copy(
                src_ref=src,
                dst_ref=comm_ref.at[h],
                send_sem=send_sems.at[h],
                recv_sem=recv_sems.at[h],
                device_id=(left,),
                device_id_type=pl.DeviceIdType.MESH,
            )
            rdma.start()
            rdma.wait()
            origin = lax.rem(my_pos + h + 1, N_DEV)
            acc = chunk_compute(comm_ref[h], origin, acc)

        out_ref[:, :] = acc

    return pl.pallas_call(
        body,
        out_shape=jax.ShapeDtypeStruct((T, H), jnp.float32),
        in_specs=[
            pl.BlockSpec(memory_space=pltpu.VMEM),
            pl.BlockSpec(memory_space=pltpu.VMEM),
            pl.BlockSpec(memory_space=pltpu.VMEM),
            pl.BlockSpec(memory_space=pltpu.VMEM),
        ],
        out_specs=pl.BlockSpec(memory_space=pltpu.VMEM),
        scratch_shapes=[
            pltpu.VMEM((N_DEV - 1, E_PER, D, H), jnp.float32),
            pltpu.SemaphoreType.DMA((N_DEV - 1,)),
            pltpu.SemaphoreType.DMA((N_DEV - 1,)),
        ],
        compiler_params=pltpu.CompilerParams(collective_id=0),
    )(x, router_W, route_idx, expert_W)


# device time: 25371 ns/iter; 1.6026x vs baseline; 1.6026x over previous
import jax
import jax.numpy as jnp
from jax import lax
from jax.experimental import pallas as pl
from jax.experimental.pallas import tpu as pltpu

N_DEV = 8
E_PER = 2
N_EXP = N_DEV * E_PER
T = 256
D = 128
H = 256


def kernel(x, router_W, route_idx, expert_W):
    def body(x_ref, rw_ref, ridx_ref, ew_ref, out_ref,
             comm_ref, send_sems, recv_sems):
        my_pos = lax.axis_index("i")

        barrier_sem = pltpu.get_barrier_semaphore()
        for k in range(1, N_DEV):
            peer = lax.rem(my_pos + k, N_DEV)
            pl.semaphore_signal(
                barrier_sem, inc=1,
                device_id=(peer,), device_id_type=pl.DeviceIdType.MESH,
            )
        pl.semaphore_wait(barrier_sem, N_DEV - 1)

        sends = []
        for k in range(1, N_DEV):
            dst = lax.rem(my_pos + k, N_DEV)
            rdma = pltpu.make_async_remote_copy(
                src_ref=ew_ref,
                dst_ref=comm_ref.at[N_DEV - 1 - k],
                send_sem=send_sems.at[k - 1],
                recv_sem=recv_sems.at[N_DEV - 1 - k],
                device_id=(dst,),
                device_id_type=pl.DeviceIdType.MESH,
            )
            rdma.start()
            sends.append(rdma)

        xv = x_ref[:, :]
        scores = jnp.dot(xv, rw_ref[:, :], preferred_element_type=jnp.float32)
        m = jnp.max(scores, axis=1, keepdims=True)
        p = jnp.exp(scores - m)
        p = p / jnp.sum(p, axis=1, keepdims=True)
        e0 = ridx_ref[:, 0:1]
        e1 = ridx_ref[:, 1:2]
        iota16 = lax.broadcasted_iota(jnp.int32, (T, N_EXP), 1)
        g0 = jnp.sum(jnp.where(iota16 == e0, p, 0.0), axis=1, keepdims=True)
        g1 = jnp.sum(jnp.where(iota16 == e1, p, 0.0), axis=1, keepdims=True)
        gs = g0 + g1
        g0n = g0 / gs
        g1n = g1 / gs

        def chunk_compute(w_pair, origin, acc):
            oa = E_PER * origin
            ca = g0n * (e0 == oa).astype(jnp.float32) \
                + g1n * (e1 == oa).astype(jnp.float32)
            cb = g0n * (e0 == oa + 1).astype(jnp.float32) \
                + g1n * (e1 == oa + 1).astype(jnp.float32)
            xcat = jnp.concatenate([ca * xv, cb * xv], axis=1)
            wcat = w_pair.reshape(E_PER * D, H)
            return acc + jnp.dot(xcat, wcat, preferred_element_type=jnp.float32)

        acc = chunk_compute(ew_ref[:, :, :], my_pos,
                            jnp.zeros((T, H), jnp.float32))

        for j in range(N_DEV - 1):
            recv = pltpu.make_async_remote_copy(
                src_ref=ew_ref,
                dst_ref=comm_ref.at[j],
                send_sem=send_sems.at[0],
                recv_sem=recv_sems.at[j],
                device_id=(my_pos,),
                device_id_type=pl.DeviceIdType.MESH,
            )
            recv.wait_recv()
            origin = lax.rem(my_pos + j + 1, N_DEV)
            acc = chunk_compute(comm_ref[j], origin, acc)

        for rdma in sends:
            rdma.wait_send()

        out_ref[:, :] = acc

    return pl.pallas_call(
        body,
        out_shape=jax.ShapeDtypeStruct((T, H), jnp.float32),
        in_specs=[
            pl.BlockSpec(memory_space=pltpu.VMEM),
            pl.BlockSpec(memory_space=pltpu.VMEM),
            pl.BlockSpec(memory_space=pltpu.VMEM),
            pl.BlockSpec(memory_space=pltpu.VMEM),
        ],
        out_specs=pl.BlockSpec(memory_space=pltpu.VMEM),
        scratch_shapes=[
            pltpu.VMEM((N_DEV - 1, E_PER, D, H), jnp.float32),
            pltpu.SemaphoreType.DMA((N_DEV - 1,)),
            pltpu.SemaphoreType.DMA((N_DEV - 1,)),
        ],
        compiler_params=pltpu.CompilerParams(collective_id=0),
    )(x, router_W, route_idx, expert_W)


# device time: 16995 ns/iter; 2.3925x vs baseline; 1.4929x over previous
import jax
import jax.numpy as jnp
from jax import lax
from jax.experimental import pallas as pl
from jax.experimental.pallas import tpu as pltpu

N_DEV = 8
E_PER = 2
N_EXP = N_DEV * E_PER
T = 256
D = 128
H = 256


def kernel(x, router_W, route_idx, expert_W):
    def body(x_ref, rw_ref, ridx_ref, ew_ref, out_ref,
             stage_ref, comm_ref, send_sems, recv_sems):
        my_pos = lax.axis_index("i")

        stage_ref[:, :, :] = ew_ref[:, :, :].astype(jnp.bfloat16)

        barrier_sem = pltpu.get_barrier_semaphore()
        for k in range(1, N_DEV):
            peer = lax.rem(my_pos + k, N_DEV)
            pl.semaphore_signal(
                barrier_sem, inc=1,
                device_id=(peer,), device_id_type=pl.DeviceIdType.MESH,
            )
        pl.semaphore_wait(barrier_sem, N_DEV - 1)

        sends = []
        for k in range(1, N_DEV):
            dst = lax.rem(my_pos + k, N_DEV)
            rdma = pltpu.make_async_remote_copy(
                src_ref=stage_ref,
                dst_ref=comm_ref.at[N_DEV - 1 - k],
                send_sem=send_sems.at[k - 1],
                recv_sem=recv_sems.at[N_DEV - 1 - k],
                device_id=(dst,),
                device_id_type=pl.DeviceIdType.MESH,
            )
            rdma.start()
            sends.append(rdma)

        xv = x_ref[:, :]
        scores = jnp.dot(xv, rw_ref[:, :], preferred_element_type=jnp.float32)
        m = jnp.max(scores, axis=1, keepdims=True)
        p = jnp.exp(scores - m)
        p = p / jnp.sum(p, axis=1, keepdims=True)
        e0 = ridx_ref[:, 0:1]
        e1 = ridx_ref[:, 1:2]
        iota16 = lax.broadcasted_iota(jnp.int32, (T, N_EXP), 1)
        g0 = jnp.sum(jnp.where(iota16 == e0, p, 0.0), axis=1, keepdims=True)
        g1 = jnp.sum(jnp.where(iota16 == e1, p, 0.0), axis=1, keepdims=True)
        gs = g0 + g1
        g0n = g0 / gs
        g1n = g1 / gs

        def chunk_compute(w_pair, origin, acc):
            oa = E_PER * origin
            ca = g0n * (e0 == oa).astype(jnp.float32) \
                + g1n * (e1 == oa).astype(jnp.float32)
            cb = g0n * (e0 == oa + 1).astype(jnp.float32) \
                + g1n * (e1 == oa + 1).astype(jnp.float32)
            xcat = jnp.concatenate(
                [ca * xv, cb * xv], axis=1).astype(jnp.bfloat16)
            wcat = w_pair.reshape(E_PER * D, H)
            return acc + jnp.dot(xcat, wcat, preferred_element_type=jnp.float32)

        acc = chunk_compute(stage_ref[:, :, :], my_pos,
                            jnp.zeros((T, H), jnp.float32))

        for j in range(N_DEV - 1):
            recv = pltpu.make_async_remote_copy(
                src_ref=stage_ref,
                dst_ref=comm_ref.at[j],
                send_sem=send_sems.at[0],
                recv_sem=recv_sems.at[j],
                device_id=(my_pos,),
                device_id_type=pl.DeviceIdType.MESH,
            )
            recv.wait_recv()
            origin = lax.rem(my_pos + j + 1, N_DEV)
            acc = chunk_compute(comm_ref[j], origin, acc)

        for rdma in sends:
            rdma.wait_send()

        out_ref[:, :] = acc

    return pl.pallas_call(
        body,
        out_shape=jax.ShapeDtypeStruct((T, H), jnp.float32),
        in_specs=[
            pl.BlockSpec(memory_space=pltpu.VMEM),
            pl.BlockSpec(memory_space=pltpu.VMEM),
            pl.BlockSpec(memory_space=pltpu.VMEM),
            pl.BlockSpec(memory_space=pltpu.VMEM),
        ],
        out_specs=pl.BlockSpec(memory_space=pltpu.VMEM),
        scratch_shapes=[
            pltpu.VMEM((E_PER, D, H), jnp.bfloat16),
            pltpu.VMEM((N_DEV - 1, E_PER, D, H), jnp.bfloat16),
            pltpu.SemaphoreType.DMA((N_DEV - 1,)),
            pltpu.SemaphoreType.DMA((N_DEV - 1,)),
        ],
        compiler_params=pltpu.CompilerParams(collective_id=0),
    )(x, router_W, route_idx, expert_W)
